# Initial kernel scaffold; baseline (speedup 1.0000x reference)
import jax
import jax.numpy as jnp
from jax.experimental import pallas as pl


def kernel(x, edge_index, batch, W_pre, b_pre, gnpre_w, gnpre_b, gnpre_ms, gn1_w, gn1_b, gn1_ms, gn2_w, gn2_b, gn2_ms, gn3_w, gn3_b, gn3_ms, gn4_w, gn4_b, gn4_ms, c1_Wl, c1_bl, c1_Wr, c1_br, c1_att, c1_Wres, c1_bias, c2_Wl, c2_bl, c2_Wr, c2_br, c2_att, c2_Wres, c2_bias, c3_Wl, c3_bl, c3_Wr, c3_br, c3_att, c3_Wres, c3_bias, c4_Wl, c4_bl, c4_Wr, c4_br, c4_att, c4_Wres, c4_bias, c5_Wl, c5_bl, c5_Wr, c5_br, c5_att, c5_Wres, c5_bias, W_o1, b_o1, W_o2, b_o2, W_cls, b_cls):
    raise NotImplementedError("write your pallas kernel here")


# speedup vs baseline: 1.0030x; 1.0030x over previous
import jax
import jax.numpy as jnp
from jax.experimental import pallas as pl

N = 10000
HID = 8
HEADS = 11
HC = HID * HEADS
NEG = 0.2


def _graph_norm(x, batch, w, b, ms):
    cnt = jnp.float32(x.shape[0])
    mean = jnp.sum(x, axis=0, keepdims=True) / cnt
    out = x - ms * mean
    var = jnp.sum(out * out, axis=0, keepdims=True) / cnt
    std = jnp.sqrt(var + 1e-5)
    return w * out / std + b


def _gatv2(x, src, dst, Wl, bl, Wr, br, att, Wres, bias, concat):
    n = x.shape[0]
    xl = (x @ Wl + bl).reshape(n, HEADS, HID)
    xr = (x @ Wr + br).reshape(n, HEADS, HID)
    m = jax.nn.leaky_relu(xl[src] + xr[dst], NEG)
    alpha = (m * att[None, :, :]).sum(-1)
    amax = jax.ops.segment_max(alpha, dst, num_segments=n)
    amax = jnp.where(jnp.isfinite(amax), amax, 0.0)
    ex = jnp.exp(alpha - amax[dst])
    den = jax.ops.segment_sum(ex, dst, num_segments=n)
    a = ex / (den[dst] + 1e-16)
    out = jax.ops.segment_sum(xl[src] * a[:, :, None], dst, num_segments=n)
    if concat:
        out = out.reshape(n, HEADS * HID)
    else:
        out = out.mean(axis=1)
    return out + x @ Wres + bias


def _mlp_body(h_ref, w1_ref, b1_ref, w2_ref, b2_ref, wc_ref, bc_ref, o_ref):
    h = h_ref[...]
    h = jax.nn.relu(h @ w1_ref[...] + b1_ref[...])
    h = jax.nn.relu(h @ w2_ref[...] + b2_ref[...])
    o_ref[...] = h @ wc_ref[...] + bc_ref[...]


def kernel(x, edge_index, batch, W_pre, b_pre, gnpre_w, gnpre_b, gnpre_ms, gn1_w, gn1_b, gn1_ms, gn2_w, gn2_b, gn2_ms, gn3_w, gn3_b, gn3_ms, gn4_w, gn4_b, gn4_ms, c1_Wl, c1_bl, c1_Wr, c1_br, c1_att, c1_Wres, c1_bias, c2_Wl, c2_bl, c2_Wr, c2_br, c2_att, c2_Wres, c2_bias, c3_Wl, c3_bl, c3_Wr, c3_br, c3_att, c3_Wres, c3_bias, c4_Wl, c4_bl, c4_Wr, c4_br, c4_att, c4_Wres, c4_bias, c5_Wl, c5_bl, c5_Wr, c5_br, c5_att, c5_Wres, c5_bias, W_o1, b_o1, W_o2, b_o2, W_cls, b_cls):
    p = dict(
        c1_Wl=c1_Wl, c1_bl=c1_bl, c1_Wr=c1_Wr, c1_br=c1_br, c1_att=c1_att, c1_Wres=c1_Wres, c1_bias=c1_bias,
        c2_Wl=c2_Wl, c2_bl=c2_bl, c2_Wr=c2_Wr, c2_br=c2_br, c2_att=c2_att, c2_Wres=c2_Wres, c2_bias=c2_bias,
        c3_Wl=c3_Wl, c3_bl=c3_bl, c3_Wr=c3_Wr, c3_br=c3_br, c3_att=c3_att, c3_Wres=c3_Wres, c3_bias=c3_bias,
        c4_Wl=c4_Wl, c4_bl=c4_bl, c4_Wr=c4_Wr, c4_br=c4_br, c4_att=c4_att, c4_Wres=c4_Wres, c4_bias=c4_bias,
    )
    src = edge_index[0]
    dst = edge_index[1]
    h = x @ W_pre + b_pre
    h = _graph_norm(h, batch, gnpre_w, gnpre_b, gnpre_ms)
    h = jax.nn.relu(h)
    for l in range(1, 5):
        h = _gatv2(h, src, dst, p["c%d_Wl" % l], p["c%d_bl" % l], p["c%d_Wr" % l],
                   p["c%d_br" % l], p["c%d_att" % l], p["c%d_Wres" % l], p["c%d_bias" % l], True)
        h = _graph_norm(h, batch, (gn1_w, gn2_w, gn3_w, gn4_w)[l - 1],
                        (gn1_b, gn2_b, gn3_b, gn4_b)[l - 1], (gn1_ms, gn2_ms, gn3_ms, gn4_ms)[l - 1])
        h = jax.nn.relu(h)
    h = _gatv2(h, src, dst, c5_Wl, c5_bl, c5_Wr, c5_br, c5_att, c5_Wres, c5_bias, False)
    h = jax.nn.relu(h)
    out = pl.pallas_call(
        _mlp_body,
        out_shape=jax.ShapeDtypeStruct((N, b_cls.shape[0]), jnp.float32),
    )(h, W_o1, b_o1, W_o2, b_o2, W_cls, b_cls)
    return out


# speedup vs baseline: 29.7884x; 29.6996x over previous
import functools

import jax
import jax.numpy as jnp
from jax import lax
from jax.experimental import pallas as pl
from jax.experimental.pallas import tpu as pltpu
from jax.experimental.pallas import tpu_sc as plsc

N = 10000
E = 320000
IN = 128
HID = 8
HEADS = 11
HC = HID * HEADS
OUTG = 64
NCLS = 16
NEG = 0.2

NV = 320
NP = 32 * NV
HCP = 96
K = 128
EP = E + 264
RPP = 31 * NV + 328


def _gn_relu(y, w, b, ms):
    mean = jnp.sum(y, axis=0, keepdims=True) * (1.0 / N)
    out = y - ms * mean
    var = jnp.sum(out * out, axis=0, keepdims=True) * (1.0 / N)
    std = jnp.sqrt(var + 1e-5)
    return jnp.maximum(w * out / std + b, 0.0)


def _proj_pad(h, W, bvec):
    y = jnp.dot(h, W, preferred_element_type=jnp.float32) + bvec
    y = jnp.concatenate([y, jnp.zeros((N, HCP - HC), jnp.float32)], axis=1)
    return jnp.concatenate([y, jnp.zeros((NP - N, HCP), jnp.float32)], axis=0)


def _pre_body(x_ref, wp_ref, bp_ref, gw_ref, gb_ref, gms_ref,
              wl_ref, bl_ref, wr_ref, br_ref, wres_ref, cb_ref,
              xl_o, xr_o, res_o):
    y = jnp.dot(x_ref[...], wp_ref[...], preferred_element_type=jnp.float32)
    y = y + bp_ref[...]
    h = _gn_relu(y, gw_ref[...], gb_ref[...], gms_ref[...])
    xl_o[...] = _proj_pad(h, wl_ref[...], bl_ref[...])
    xr_o[...] = _proj_pad(h, wr_ref[...], br_ref[...])
    res_o[...] = jnp.dot(h, wres_ref[...], preferred_element_type=jnp.float32) + cb_ref[...]


def _mid_body(sc_ref, res_ref, gw_ref, gb_ref, gms_ref,
              wl_ref, bl_ref, wr_ref, br_ref, wres_ref, cb_ref,
              xl_o, xr_o, res_o):
    g = sc_ref[0:N, 0:HC] + res_ref[...]
    h = _gn_relu(g, gw_ref[...], gb_ref[...], gms_ref[...])
    xl_o[...] = _proj_pad(h, wl_ref[...], bl_ref[...])
    xr_o[...] = _proj_pad(h, wr_ref[...], br_ref[...])
    res_o[...] = jnp.dot(h, wres_ref[...], preferred_element_type=jnp.float32) + cb_ref[...]


def _post_body(sc_ref, res_ref, w1_ref, b1_ref, w2_ref, b2_ref, wc_ref, bc_ref, o_ref):
    s = sc_ref[0:N, 0:HC]
    m = jnp.zeros((N, HID), jnp.float32)
    for h in range(HEADS):
        m = m + s[:, HID * h:HID * (h + 1)]
    g = jnp.maximum(m * (1.0 / HEADS) + res_ref[...], 0.0)
    h1 = jnp.maximum(jnp.dot(g, w1_ref[...], preferred_element_type=jnp.float32) + b1_ref[...], 0.0)
    h2 = jnp.maximum(jnp.dot(h1, w2_ref[...], preferred_element_type=jnp.float32) + b2_ref[...], 0.0)
    o_ref[...] = jnp.dot(h2, wc_ref[...], preferred_element_type=jnp.float32) + bc_ref[...]


def _tc_pre(x, wp, bp, gw, gb, gms, wl, bl, wr, br, wres, cb):
    return pl.pallas_call(
        _pre_body,
        out_shape=[
            jax.ShapeDtypeStruct((NP, HCP), jnp.float32),
            jax.ShapeDtypeStruct((NP, HCP), jnp.float32),
            jax.ShapeDtypeStruct((N, wres.shape[1]), jnp.float32),
        ],
    )(x, wp, bp, gw, gb, gms, wl, bl, wr, br, wres, cb)


def _tc_mid(sc, res, gw, gb, gms, wl, bl, wr, br, wres, cb):
    return pl.pallas_call(
        _mid_body,
        out_shape=[
            jax.ShapeDtypeStruct((NP, HCP), jnp.float32),
            jax.ShapeDtypeStruct((NP, HCP), jnp.float32),
            jax.ShapeDtypeStruct((N, wres.shape[1]), jnp.float32),
        ],
    )(sc, res, gw, gb, gms, wl, bl, wr, br, wres, cb)


def _tc_post(sc, res, w1, b1, w2, b2, wc, bc):
    return pl.pallas_call(
        _post_body,
        out_shape=jax.ShapeDtypeStruct((N, NCLS), jnp.float32),
    )(sc, res, w1, b1, w2, b2, wc, bc)


@functools.cache
def _gat_edge_kernel():
    mesh = plsc.VectorSubcoreMesh(core_axis_name="c", subcore_axis_name="s",
                                  num_cores=2, num_subcores=16)
    return functools.partial(
        pl.kernel,
        out_type=jax.ShapeDtypeStruct((NP, HCP), jnp.float32),
        mesh=mesh,
        compiler_params=pltpu.CompilerParams(needs_layout_passes=False,
                                             use_tc_tiling_on_sc=False),
        scratch_types=_gat_scratch(),
    )(_gat_edge_body)


def _gat_edge(xl, xr, srcp, rpp, att):
    return _gat_edge_kernel()(xl, xr, srcp, rpp, att)


def _gat_scratch():
    return [
        pltpu.VMEM((328,), jnp.int32),
        pltpu.VMEM((HCP,), jnp.float32),
        pltpu.VMEM((K,), jnp.int32),
        pltpu.VMEM((K,), jnp.int32),
        pltpu.VMEM((2 * K, HCP), jnp.float32),
        pltpu.VMEM((NV, HCP), jnp.float32),
        pltpu.VMEM((NV, HCP), jnp.float32),
        pltpu.VMEM((128,), jnp.float32),
        pltpu.VMEM((32,), jnp.float32),
        pltpu.SemaphoreType.DMA,
        pltpu.SemaphoreType.DMA,
    ]


def _gat_edge_body(xl_hbm, xr_hbm, src_hbm, rp_hbm, att_hbm, out_hbm,
              rp_v, att_v, idx0, idx1, rows_v, xr_v, out_v, prod_v, sc_v,
              sem0, sem1):
    wid = lax.axis_index("s") * 2 + lax.axis_index("c")
    v0 = pl.multiple_of(wid * NV, 8)
    pltpu.sync_copy(rp_hbm.at[pl.ds(v0, 328)], rp_v)
    pltpu.sync_copy(xr_hbm.at[pl.ds(v0, NV)], xr_v)
    pltpu.sync_copy(att_hbm, att_v)

    iota = lax.iota(jnp.int32, 16)
    cols = [iota + 16 * j for j in range(6)]
    iota8 = iota * 8
    expd = [(iota >> 3) + 2 * j for j in range(6)]
    z16 = jnp.zeros((16,), jnp.float32)

    def sread(i):
        return jnp.max(plsc.load_gather(rp_v, [jnp.full((16,), i, jnp.int32)]))

    e0 = jnp.max(jnp.where(iota == 0, rp_v[pl.ds(0, 16)],
                           jnp.int32(-2147483647)))
    ea0 = pl.multiple_of((e0 >> 3) << 3, 8)
    shift = e0 - ea0

    for jj in range(8):
        prod_v[pl.ds(16 * jj, 16)] = z16

    rows0 = rows_v.at[pl.ds(0, K)]
    rows1 = rows_v.at[pl.ds(K, K)]

    def issue(start, idxb, rowsb, semb):
        pltpu.sync_copy(src_hbm.at[pl.ds(pl.multiple_of(start, 8), K)], idxb)
        pltpu.async_copy(xl_hbm.at[idxb], rowsb, semb)

    def wait(idxb, rowsb, semb):
        pltpu.make_async_copy(xl_hbm.at[idxb], rowsb, semb).wait()

    issue(ea0, idx0, rows0, sem0)
    wait(idx0, rows0, sem0)
    issue(ea0 + K, idx1, rows1, sem1)

    att_r = [att_v[pl.ds(16 * j, 16)] for j in range(6)]
    minit = jnp.full((16,), -1e30, jnp.float32)

    def node_body(vi, b0):
        b1 = sread(vi + 1)
        vis = jnp.full((16,), vi, jnp.int32)
        xr_r = [plsc.load_gather(xr_v, [vis, cols[j]]) for j in range(6)]

        def edge_body(i, st):
            m, d = st[0], st[1]
            acc = st[2:]
            le = (b0 - ea0) + i
            cross = jnp.logical_and((le & (K - 1)) == 0, le > shift)
            c = le >> 7
            par = c & 1

            @pl.when(jnp.logical_and(cross, par == 0))
            def _():
                wait(idx0, rows0, sem0)
                issue(ea0 + (c + 1) * K, idx1, rows1, sem1)

            @pl.when(jnp.logical_and(cross, par == 1))
            def _():
                wait(idx1, rows1, sem1)
                issue(ea0 + (c + 1) * K, idx0, rows0, sem0)

            row = jnp.full((16,), le & (2 * K - 1), jnp.int32)
            xlr = []
            for j in range(6):
                xlj = plsc.load_gather(rows_v, [row, cols[j]])
                xlr.append(xlj)
                t = xlj + xr_r[j]
                t = jnp.maximum(t, t * NEG)
                prod_v[pl.ds(16 * j, 16)] = t * att_r[j]
            alpha = z16
            for dd in range(8):
                alpha = alpha + plsc.load_gather(prod_v, [iota8 + dd])
            mn = jnp.maximum(m, alpha)
            corr = jnp.exp(m - mn)
            p = jnp.exp(alpha - mn)
            dn = d * corr + p
            sc_v[pl.ds(0, 16)] = corr
            sc_v[pl.ds(16, 16)] = p
            nacc = []
            for j in range(6):
                ce = plsc.load_gather(sc_v, [expd[j]])
                pe = plsc.load_gather(sc_v, [expd[j] + 16])
                nacc.append(acc[j] * ce + pe * xlr[j])
            return (mn, dn) + tuple(nacc)

        st = lax.fori_loop(0, b1 - b0, edge_body,
                           (minit, z16, z16, z16, z16, z16, z16, z16))
        rcp = 1.0 / (st[1] + 1e-16)
        sc_v[pl.ds(0, 16)] = rcp
        for j in range(6):
            re = plsc.load_gather(sc_v, [expd[j]])
            plsc.store_scatter(out_v, [vis, cols[j]], st[2 + j] * re)
        return b1

    lax.fori_loop(0, NV, node_body, e0)

    e1 = jnp.max(jnp.where(iota == NV - 312, rp_v[pl.ds(312, 16)],
                           jnp.int32(-2147483647)))
    total = shift + (e1 - e0)
    c_out = jnp.maximum(1, ((total - 1) >> 7) + 1)

    @pl.when((c_out & 1) == 0)
    def _():
        wait(idx0, rows0, sem0)

    @pl.when((c_out & 1) == 1)
    def _():
        wait(idx1, rows1, sem1)

    pltpu.sync_copy(out_v, out_hbm.at[pl.ds(v0, NV)])


def kernel(x, edge_index, batch, W_pre, b_pre, gnpre_w, gnpre_b, gnpre_ms, gn1_w, gn1_b, gn1_ms, gn2_w, gn2_b, gn2_ms, gn3_w, gn3_b, gn3_ms, gn4_w, gn4_b, gn4_ms, c1_Wl, c1_bl, c1_Wr, c1_br, c1_att, c1_Wres, c1_bias, c2_Wl, c2_bl, c2_Wr, c2_br, c2_att, c2_Wres, c2_bias, c3_Wl, c3_bl, c3_Wr, c3_br, c3_att, c3_Wres, c3_bias, c4_Wl, c4_bl, c4_Wr, c4_br, c4_att, c4_Wres, c4_bias, c5_Wl, c5_bl, c5_Wr, c5_br, c5_att, c5_Wres, c5_bias, W_o1, b_o1, W_o2, b_o2, W_cls, b_cls):
    src = edge_index[0]
    dst = edge_index[1]
    perm = jnp.argsort(dst)
    src_s = jnp.take(src, perm).astype(jnp.int32)
    dst_s = jnp.take(dst, perm)
    rp = jnp.searchsorted(dst_s, jnp.arange(N + 1), side="left").astype(jnp.int32)
    rp_pad = jnp.concatenate([rp, jnp.full((RPP - N - 1,), E, jnp.int32)])
    src_pad = jnp.concatenate([src_s, jnp.zeros((EP - E,), jnp.int32)])

    def att96(a):
        return jnp.concatenate([a.reshape(HC), jnp.zeros((HCP - HC,), jnp.float32)])

    gn = ((gn1_w, gn1_b, gn1_ms), (gn2_w, gn2_b, gn2_ms),
          (gn3_w, gn3_b, gn3_ms), (gn4_w, gn4_b, gn4_ms))
    layers = ((c1_Wl, c1_bl, c1_Wr, c1_br, c1_att, c1_Wres, c1_bias),
              (c2_Wl, c2_bl, c2_Wr, c2_br, c2_att, c2_Wres, c2_bias),
              (c3_Wl, c3_bl, c3_Wr, c3_br, c3_att, c3_Wres, c3_bias),
              (c4_Wl, c4_bl, c4_Wr, c4_br, c4_att, c4_Wres, c4_bias),
              (c5_Wl, c5_bl, c5_Wr, c5_br, c5_att, c5_Wres, c5_bias))

    xl, xr, res = _tc_pre(x, W_pre, b_pre, gnpre_w, gnpre_b, gnpre_ms,
                          layers[0][0], layers[0][1], layers[0][2], layers[0][3],
                          layers[0][5], layers[0][6])
    for l in range(4):
        sc = _gat_edge(xl, xr, src_pad, rp_pad, att96(layers[l][4]))
        nxt = layers[l + 1]
        xl, xr, res = _tc_mid(sc, res, gn[l][0], gn[l][1], gn[l][2],
                              nxt[0], nxt[1], nxt[2], nxt[3], nxt[5], nxt[6])
    sc = _gat_edge(xl, xr, src_pad, rp_pad, att96(layers[4][4]))
    return _tc_post(sc, res, W_o1, b_o1, W_o2, b_o2, W_cls, b_cls)


# speedup vs baseline: 52.2329x; 1.7535x over previous
import functools

import jax
import jax.numpy as jnp
from jax import lax
from jax.experimental import pallas as pl
from jax.experimental.pallas import tpu as pltpu
from jax.experimental.pallas import tpu_sc as plsc

N = 10000
E = 320000
IN = 128
HID = 8
HEADS = 11
HC = HID * HEADS
OUTG = 64
NCLS = 16
NEG = 0.2

NV = 320
NP = 32 * NV
HCP = 96
K = 128
EP = E + 264
RPP = 31 * NV + 328


def _gn_relu(y, w, b, ms):
    mean = jnp.sum(y, axis=0, keepdims=True) * (1.0 / N)
    out = y - ms * mean
    var = jnp.sum(out * out, axis=0, keepdims=True) * (1.0 / N)
    std = jnp.sqrt(var + 1e-5)
    return jnp.maximum(w * out / std + b, 0.0)


def _proj_pad(h, W, bvec):
    y = jnp.dot(h, W, preferred_element_type=jnp.float32) + bvec
    y = jnp.concatenate([y, jnp.zeros((N, HCP - HC), jnp.float32)], axis=1)
    return jnp.concatenate([y, jnp.zeros((NP - N, HCP), jnp.float32)], axis=0)


def _pre_body(x_ref, wp_ref, bp_ref, gw_ref, gb_ref, gms_ref,
              wl_ref, bl_ref, wr_ref, br_ref, wres_ref, cb_ref,
              xl_o, xr_o, res_o):
    y = jnp.dot(x_ref[...], wp_ref[...], preferred_element_type=jnp.float32)
    y = y + bp_ref[...]
    h = _gn_relu(y, gw_ref[...], gb_ref[...], gms_ref[...])
    xl_o[...] = _proj_pad(h, wl_ref[...], bl_ref[...])
    xr_o[...] = _proj_pad(h, wr_ref[...], br_ref[...])
    res_o[...] = jnp.dot(h, wres_ref[...], preferred_element_type=jnp.float32) + cb_ref[...]


def _mid_body(sc_ref, res_ref, gw_ref, gb_ref, gms_ref,
              wl_ref, bl_ref, wr_ref, br_ref, wres_ref, cb_ref,
              xl_o, xr_o, res_o):
    g = sc_ref[0:N, 0:HC] + res_ref[...]
    h = _gn_relu(g, gw_ref[...], gb_ref[...], gms_ref[...])
    xl_o[...] = _proj_pad(h, wl_ref[...], bl_ref[...])
    xr_o[...] = _proj_pad(h, wr_ref[...], br_ref[...])
    res_o[...] = jnp.dot(h, wres_ref[...], preferred_element_type=jnp.float32) + cb_ref[...]


def _post_body(sc_ref, res_ref, w1_ref, b1_ref, w2_ref, b2_ref, wc_ref, bc_ref, o_ref):
    s = sc_ref[0:N, 0:HC]
    m = jnp.zeros((N, HID), jnp.float32)
    for h in range(HEADS):
        m = m + s[:, HID * h:HID * (h + 1)]
    g = jnp.maximum(m * (1.0 / HEADS) + res_ref[...], 0.0)
    h1 = jnp.maximum(jnp.dot(g, w1_ref[...], preferred_element_type=jnp.float32) + b1_ref[...], 0.0)
    h2 = jnp.maximum(jnp.dot(h1, w2_ref[...], preferred_element_type=jnp.float32) + b2_ref[...], 0.0)
    o_ref[...] = jnp.dot(h2, wc_ref[...], preferred_element_type=jnp.float32) + bc_ref[...]


def _tc_pre(x, wp, bp, gw, gb, gms, wl, bl, wr, br, wres, cb):
    return pl.pallas_call(
        _pre_body,
        out_shape=[
            jax.ShapeDtypeStruct((NP, HCP), jnp.float32),
            jax.ShapeDtypeStruct((NP, HCP), jnp.float32),
            jax.ShapeDtypeStruct((N, wres.shape[1]), jnp.float32),
        ],
    )(x, wp, bp, gw, gb, gms, wl, bl, wr, br, wres, cb)


def _tc_mid(sc, res, gw, gb, gms, wl, bl, wr, br, wres, cb):
    return pl.pallas_call(
        _mid_body,
        out_shape=[
            jax.ShapeDtypeStruct((NP, HCP), jnp.float32),
            jax.ShapeDtypeStruct((NP, HCP), jnp.float32),
            jax.ShapeDtypeStruct((N, wres.shape[1]), jnp.float32),
        ],
    )(sc, res, gw, gb, gms, wl, bl, wr, br, wres, cb)


def _tc_post(sc, res, w1, b1, w2, b2, wc, bc):
    return pl.pallas_call(
        _post_body,
        out_shape=jax.ShapeDtypeStruct((N, NCLS), jnp.float32),
    )(sc, res, w1, b1, w2, b2, wc, bc)


@functools.cache
def _gat_edge_kernel():
    mesh = plsc.VectorSubcoreMesh(core_axis_name="c", subcore_axis_name="s",
                                  num_cores=2, num_subcores=16)
    return functools.partial(
        pl.kernel,
        out_type=jax.ShapeDtypeStruct((NP, HCP), jnp.float32),
        mesh=mesh,
        compiler_params=pltpu.CompilerParams(needs_layout_passes=False,
                                             use_tc_tiling_on_sc=False),
        scratch_types=_gat_scratch(),
    )(_gat_edge_body)


def _gat_edge(xl, xr, srcp, rpp, att):
    return _gat_edge_kernel()(xl, xr, srcp, rpp, att)


def _gat_scratch():
    return [
        pltpu.VMEM((328,), jnp.int32),
        pltpu.VMEM((HCP,), jnp.float32),
        pltpu.VMEM((K,), jnp.int32),
        pltpu.VMEM((K,), jnp.int32),
        pltpu.VMEM((2 * K, HCP), jnp.float32),
        pltpu.VMEM((NV, HCP), jnp.float32),
        pltpu.VMEM((NV, HCP), jnp.float32),
        pltpu.VMEM((128,), jnp.float32),
        pltpu.VMEM((32,), jnp.float32),
        pltpu.SemaphoreType.DMA,
        pltpu.SemaphoreType.DMA,
    ]


def _gat_edge_body(xl_hbm, xr_hbm, src_hbm, rp_hbm, att_hbm, out_hbm,
              rp_v, att_v, idx0, idx1, rows_v, xr_v, out_v, prod_v, sc_v,
              sem0, sem1):
    wid = lax.axis_index("s") * 2 + lax.axis_index("c")
    v0 = pl.multiple_of(wid * NV, 8)
    pltpu.sync_copy(rp_hbm.at[pl.ds(v0, 328)], rp_v)
    pltpu.sync_copy(xr_hbm.at[pl.ds(v0, NV)], xr_v)
    pltpu.sync_copy(att_hbm, att_v)

    iota = lax.iota(jnp.int32, 16)
    cols = [iota + 16 * j for j in range(6)]
    iota8 = iota * 8
    expd = [(iota >> 3) + 2 * j for j in range(6)]
    z16 = jnp.zeros((16,), jnp.float32)

    def sread(i):
        return jnp.max(plsc.load_gather(rp_v, [jnp.full((16,), i, jnp.int32)]))

    e0 = jnp.max(jnp.where(iota == 0, rp_v[pl.ds(0, 16)],
                           jnp.int32(-2147483647)))
    ea0 = pl.multiple_of((e0 >> 3) << 3, 8)
    shift = e0 - ea0

    for jj in range(8):
        prod_v[pl.ds(16 * jj, 16)] = z16

    rows0 = rows_v.at[pl.ds(0, K)]
    rows1 = rows_v.at[pl.ds(K, K)]

    def issue(start, idxb, rowsb, semb):
        pltpu.sync_copy(src_hbm.at[pl.ds(pl.multiple_of(start, 8), K)], idxb)
        pltpu.async_copy(xl_hbm.at[idxb], rowsb, semb)

    def wait(idxb, rowsb, semb):
        pltpu.make_async_copy(xl_hbm.at[idxb], rowsb, semb).wait()

    issue(ea0, idx0, rows0, sem0)
    wait(idx0, rows0, sem0)
    issue(ea0 + K, idx1, rows1, sem1)

    att_r = [att_v[pl.ds(16 * j, 16)] for j in range(6)]
    minit = jnp.full((16,), -1e30, jnp.float32)

    def node_body(vi, b0):
        b1 = sread(vi + 1)
        vis = jnp.full((16,), vi, jnp.int32)
        xr_r = [plsc.load_gather(xr_v, [vis, cols[j]]) for j in range(6)]

        def seg_cond(st):
            return st[0] < b1

        def seg_body(st):
            lo = st[0]
            lc = lo - ea0
            c = lc >> 7
            cross = jnp.logical_and((lc & (K - 1)) == 0, lc > shift)
            par = c & 1

            @pl.when(jnp.logical_and(cross, par == 0))
            def _():
                wait(idx0, rows0, sem0)
                issue(ea0 + (c + 1) * K, idx1, rows1, sem1)

            @pl.when(jnp.logical_and(cross, par == 1))
            def _():
                wait(idx1, rows1, sem1)
                issue(ea0 + (c + 1) * K, idx0, rows0, sem0)

            hi = jnp.minimum(b1, ea0 + (c + 1) * K)
            row0 = jnp.full((16,), lc & (2 * K - 1), jnp.int32)

            def edge_body(i, est):
                rowv, em, ed = est[0], est[1], est[2]
                eacc = est[3:]
                xlr = [plsc.load_gather(rows_v, [rowv, cols[j]])
                       for j in range(6)]
                tt = [xlr[j] + xr_r[j] for j in range(6)]
                tt = [jnp.maximum(tt[j], tt[j] * NEG) for j in range(6)]
                pp = [tt[j] * att_r[j] for j in range(6)]
                for j in range(6):
                    prod_v[pl.ds(16 * j, 16)] = pp[j]
                g = [plsc.load_gather(prod_v, [iota8 + dd]) for dd in range(8)]
                alpha = ((g[0] + g[1]) + (g[2] + g[3])) + ((g[4] + g[5]) + (g[6] + g[7]))
                mn = jnp.maximum(em, alpha)
                corr = jnp.exp(em - mn)
                p = jnp.exp(alpha - mn)
                dn = ed * corr + p
                sc_v[pl.ds(0, 16)] = corr
                sc_v[pl.ds(16, 16)] = p
                ce = [plsc.load_gather(sc_v, [expd[j]]) for j in range(6)]
                pe = [plsc.load_gather(sc_v, [expd[j] + 16]) for j in range(6)]
                nacc = [eacc[j] * ce[j] + pe[j] * xlr[j] for j in range(6)]
                return (rowv + 1, mn, dn) + tuple(nacc)

            est = lax.fori_loop(0, hi - lo, edge_body, (row0,) + tuple(st[1:]))
            return (hi,) + tuple(est[1:])

        st = lax.while_loop(seg_cond, seg_body,
                            (b0, minit, z16, z16, z16, z16, z16, z16, z16))
        rcp = 1.0 / (st[2] + 1e-16)
        sc_v[pl.ds(0, 16)] = rcp
        for j in range(6):
            re = plsc.load_gather(sc_v, [expd[j]])
            plsc.store_scatter(out_v, [vis, cols[j]], st[3 + j] * re)
        return b1

    lax.fori_loop(0, NV, node_body, e0)

    e1 = jnp.max(jnp.where(iota == NV - 312, rp_v[pl.ds(312, 16)],
                           jnp.int32(-2147483647)))
    total = shift + (e1 - e0)
    c_out = jnp.maximum(1, ((total - 1) >> 7) + 1)

    @pl.when((c_out & 1) == 0)
    def _():
        wait(idx0, rows0, sem0)

    @pl.when((c_out & 1) == 1)
    def _():
        wait(idx1, rows1, sem1)

    pltpu.sync_copy(out_v, out_hbm.at[pl.ds(v0, NV)])


def kernel(x, edge_index, batch, W_pre, b_pre, gnpre_w, gnpre_b, gnpre_ms, gn1_w, gn1_b, gn1_ms, gn2_w, gn2_b, gn2_ms, gn3_w, gn3_b, gn3_ms, gn4_w, gn4_b, gn4_ms, c1_Wl, c1_bl, c1_Wr, c1_br, c1_att, c1_Wres, c1_bias, c2_Wl, c2_bl, c2_Wr, c2_br, c2_att, c2_Wres, c2_bias, c3_Wl, c3_bl, c3_Wr, c3_br, c3_att, c3_Wres, c3_bias, c4_Wl, c4_bl, c4_Wr, c4_br, c4_att, c4_Wres, c4_bias, c5_Wl, c5_bl, c5_Wr, c5_br, c5_att, c5_Wres, c5_bias, W_o1, b_o1, W_o2, b_o2, W_cls, b_cls):
    src = edge_index[0]
    dst = edge_index[1]
    dst_s, src_s = lax.sort((dst, src), num_keys=1)
    src_s = src_s.astype(jnp.int32)
    rp = jnp.searchsorted(dst_s, jnp.arange(N + 1), side="left").astype(jnp.int32)
    rp_pad = jnp.concatenate([rp, jnp.full((RPP - N - 1,), E, jnp.int32)])
    src_pad = jnp.concatenate([src_s, jnp.zeros((EP - E,), jnp.int32)])

    def att96(a):
        return jnp.concatenate([a.reshape(HC), jnp.zeros((HCP - HC,), jnp.float32)])

    gn = ((gn1_w, gn1_b, gn1_ms), (gn2_w, gn2_b, gn2_ms),
          (gn3_w, gn3_b, gn3_ms), (gn4_w, gn4_b, gn4_ms))
    layers = ((c1_Wl, c1_bl, c1_Wr, c1_br, c1_att, c1_Wres, c1_bias),
              (c2_Wl, c2_bl, c2_Wr, c2_br, c2_att, c2_Wres, c2_bias),
              (c3_Wl, c3_bl, c3_Wr, c3_br, c3_att, c3_Wres, c3_bias),
              (c4_Wl, c4_bl, c4_Wr, c4_br, c4_att, c4_Wres, c4_bias),
              (c5_Wl, c5_bl, c5_Wr, c5_br, c5_att, c5_Wres, c5_bias))

    xl, xr, res = _tc_pre(x, W_pre, b_pre, gnpre_w, gnpre_b, gnpre_ms,
                          layers[0][0], layers[0][1], layers[0][2], layers[0][3],
                          layers[0][5], layers[0][6])
    for l in range(4):
        sc = _gat_edge(xl, xr, src_pad, rp_pad, att96(layers[l][4]))
        nxt = layers[l + 1]
        xl, xr, res = _tc_mid(sc, res, gn[l][0], gn[l][1], gn[l][2],
                              nxt[0], nxt[1], nxt[2], nxt[3], nxt[5], nxt[6])
    sc = _gat_edge(xl, xr, src_pad, rp_pad, att96(layers[4][4]))
    return _tc_post(sc, res, W_o1, b_o1, W_o2, b_o2, W_cls, b_cls)
